# baseline (speedup 1.0000x reference)
import functools

import jax
import jax.numpy as jnp
from jax.experimental import pallas as pl
from jax.experimental.pallas import tpu as pltpu

N = 10000
E = 320000
NODE_DIM = 128
EDGE_DIM = 16
HIDDEN = 2 * NODE_DIM
N_LAYERS = 3
N_GRAPHS = 16

EBLK = 2000


def _mlp_body(hi_ref, hj_ref, ea_ref, w1a_ref, w1b_ref, w1c_ref, b1_ref,
              w2_ref, b2_ref, out_ref):
    acc = jnp.dot(hi_ref[...], w1a_ref[...], preferred_element_type=jnp.float32)
    acc += jnp.dot(hj_ref[...], w1b_ref[...], preferred_element_type=jnp.float32)
    acc += jnp.dot(ea_ref[...], w1c_ref[...], preferred_element_type=jnp.float32)
    acc += b1_ref[...]
    s = acc * jax.nn.sigmoid(acc)
    out = jnp.dot(s, w2_ref[...], preferred_element_type=jnp.float32)
    out_ref[...] = out + b2_ref[...]


@jax.jit
def _mlp(h_i, h_j, ea, w1, b1, w2, b2):
    w1a = w1[:NODE_DIM]
    w1b = w1[NODE_DIM:2 * NODE_DIM]
    w1c = w1[2 * NODE_DIM:]
    grid = (E // EBLK,)
    full = lambda shape: pl.BlockSpec(shape, lambda i: (0,) * len(shape))
    return pl.pallas_call(
        _mlp_body,
        grid=grid,
        in_specs=[
            pl.BlockSpec((EBLK, NODE_DIM), lambda i: (i, 0)),
            pl.BlockSpec((EBLK, NODE_DIM), lambda i: (i, 0)),
            pl.BlockSpec((EBLK, EDGE_DIM), lambda i: (i, 0)),
            full((NODE_DIM, HIDDEN)),
            full((NODE_DIM, HIDDEN)),
            full((EDGE_DIM, HIDDEN)),
            full((1, HIDDEN)),
            full((HIDDEN, NODE_DIM)),
            full((1, NODE_DIM)),
        ],
        out_specs=pl.BlockSpec((EBLK, NODE_DIM), lambda i: (i, 0)),
        out_shape=jax.ShapeDtypeStruct((E, NODE_DIM), jnp.float32),
    )(h_i, h_j, ea, w1a, w1b, w1c, b1.reshape(1, HIDDEN), w2,
      b2.reshape(1, NODE_DIM))


def kernel(atoms, edge_index, coordinates, is_receptor, batch, node_table,
           subunit_table, W1s, b1s, W2s, b2s, fc_w, fc_b):
    src = edge_index[0]
    dst = edge_index[1]
    h = jnp.take(node_table, atoms, axis=0)
    diff = coordinates[src] - coordinates[dst]
    dist = jnp.sqrt(jnp.sum(diff * diff, axis=1) + 1e-12)
    offset = jnp.linspace(0.0, 5.0, EDGE_DIM // 2)
    coeff = -0.5 / (offset[1] - offset[0]) ** 2
    rbf = jnp.exp(coeff * jnp.square(dist[:, None] - offset[None, :]))
    is_inter = (is_receptor[src] != is_receptor[dst]).astype(jnp.int32)
    edge_kind = jnp.take(subunit_table, is_inter, axis=0)
    edge_attr = jnp.concatenate([rbf, edge_kind], axis=1)
    for i in range(N_LAYERS):
        h_i = h[dst]
        h_j = h[src]
        m = _mlp(h_i, h_j, edge_attr, W1s[i], b1s[i], W2s[i], b2s[i])
        agg = jax.ops.segment_sum(m, dst, num_segments=N)
        h = jax.nn.relu(h + agg)
    sums = jax.ops.segment_sum(h, batch, num_segments=N_GRAPHS)
    counts = jax.ops.segment_sum(jnp.ones((N,), dtype=h.dtype), batch,
                                 num_segments=N_GRAPHS)
    pooled = sums / jnp.clip(counts, 1.0)[:, None]
    out = pooled @ fc_w + fc_b
    return out.squeeze(-1)


# speedup vs baseline: 1.5647x; 1.5647x over previous
import functools

import jax
import jax.numpy as jnp
from jax import lax
from jax.experimental import pallas as pl
from jax.experimental.pallas import tpu as pltpu
from jax.experimental.pallas import tpu_sc as plsc

N = 10000
E = 320000
NODE_DIM = 128
EDGE_DIM = 16
HIDDEN = 2 * NODE_DIM
N_LAYERS = 3
N_GRAPHS = 16

EBLK = 2000

_NC = 2
_NS = 16
_NW = _NC * _NS
_EPW = E // _NW
_GC = 400

_sc_mesh = lambda: plsc.VectorSubcoreMesh(core_axis_name="c",
                                          subcore_axis_name="s")


@functools.partial(
    pl.kernel,
    out_type=(jax.ShapeDtypeStruct((E, NODE_DIM), jnp.float32),
              jax.ShapeDtypeStruct((E, NODE_DIM), jnp.float32)),
    mesh=_sc_mesh(),
    scratch_types=[
        pltpu.VMEM((_GC,), jnp.int32),
        pltpu.VMEM((_GC, NODE_DIM), jnp.float32),
        pltpu.VMEM((_GC,), jnp.int32),
        pltpu.VMEM((_GC, NODE_DIM), jnp.float32),
        pltpu.SemaphoreType.DMA,
        pltpu.SemaphoreType.DMA,
    ],
)
def _sc_gather2(h_hbm, src_hbm, dst_hbm, hj_hbm, hi_hbm,
                sidx, srows, didx, drows, sem_s, sem_d):
    wid = lax.axis_index("s") * _NC + lax.axis_index("c")
    base0 = wid * _EPW

    def body(it, carry):
        base = base0 + it * _GC
        pltpu.sync_copy(src_hbm.at[pl.ds(base, _GC)], sidx)
        pltpu.sync_copy(dst_hbm.at[pl.ds(base, _GC)], didx)
        cp_s = pltpu.async_copy(h_hbm.at[sidx], srows, sem_s)
        cp_d = pltpu.async_copy(h_hbm.at[didx], drows, sem_d)
        cp_s.wait()
        cp_d.wait()
        pltpu.sync_copy(srows, hj_hbm.at[pl.ds(base, _GC)])
        pltpu.sync_copy(drows, hi_hbm.at[pl.ds(base, _GC)])
        return carry

    lax.fori_loop(0, _EPW // _GC, body, 0)


_SCC = 80
_SCI = _EPW // _SCC
_NP = 10240
_NPT = _NP // _NS


@functools.partial(
    pl.kernel,
    out_type=jax.ShapeDtypeStruct((_NC, _NP, NODE_DIM), jnp.float32),
    mesh=_sc_mesh(),
    scratch_types=[
        pltpu.VMEM((_SCI, _SCC), jnp.int32),
        pltpu.VMEM((_SCC, NODE_DIM), jnp.float32),
        pltpu.VMEM_SHARED((_NP, NODE_DIM), jnp.float32),
    ],
)
def _sc_scatter_add(m_hbm, dst3_hbm, zeros_hbm, out_hbm, idx_v, rows_v, acc_sh):
    cid = lax.axis_index("c")
    sid = lax.axis_index("s")
    wid = sid * _NC + cid
    base0 = wid * _EPW
    pltpu.sync_copy(zeros_hbm.at[pl.ds(sid * _NPT, _NPT)],
                    acc_sh.at[pl.ds(sid * _NPT, _NPT)])
    pltpu.sync_copy(dst3_hbm.at[wid], idx_v)
    plsc.subcore_barrier()

    def body(j, carry):
        pltpu.sync_copy(m_hbm.at[pl.ds(base0 + j * _SCC, _SCC)], rows_v)
        pltpu.sync_copy(rows_v, acc_sh.at[idx_v.at[j]], add=True)
        return carry

    lax.fori_loop(0, _SCI, body, 0)
    plsc.subcore_barrier()
    pltpu.sync_copy(acc_sh.at[pl.ds(sid * _NPT, _NPT)],
                    out_hbm.at[cid, pl.ds(sid * _NPT, _NPT)])


def _mlp_body(hi_ref, hj_ref, ea_ref, w1a_ref, w1b_ref, w1c_ref, b1_ref,
              w2_ref, b2_ref, out_ref):
    acc = jnp.dot(hi_ref[...], w1a_ref[...], preferred_element_type=jnp.float32)
    acc += jnp.dot(hj_ref[...], w1b_ref[...], preferred_element_type=jnp.float32)
    acc += jnp.dot(ea_ref[...], w1c_ref[...], preferred_element_type=jnp.float32)
    acc += b1_ref[...]
    s = acc * jax.nn.sigmoid(acc)
    out = jnp.dot(s, w2_ref[...], preferred_element_type=jnp.float32)
    out_ref[...] = out + b2_ref[...]


@jax.jit
def _mlp(h_i, h_j, ea, w1, b1, w2, b2):
    w1a = w1[:NODE_DIM]
    w1b = w1[NODE_DIM:2 * NODE_DIM]
    w1c = w1[2 * NODE_DIM:]
    grid = (E // EBLK,)
    full = lambda shape: pl.BlockSpec(shape, lambda i: (0,) * len(shape))
    return pl.pallas_call(
        _mlp_body,
        grid=grid,
        in_specs=[
            pl.BlockSpec((EBLK, NODE_DIM), lambda i: (i, 0)),
            pl.BlockSpec((EBLK, NODE_DIM), lambda i: (i, 0)),
            pl.BlockSpec((EBLK, EDGE_DIM), lambda i: (i, 0)),
            full((NODE_DIM, HIDDEN)),
            full((NODE_DIM, HIDDEN)),
            full((EDGE_DIM, HIDDEN)),
            full((1, HIDDEN)),
            full((HIDDEN, NODE_DIM)),
            full((1, NODE_DIM)),
        ],
        out_specs=pl.BlockSpec((EBLK, NODE_DIM), lambda i: (i, 0)),
        out_shape=jax.ShapeDtypeStruct((E, NODE_DIM), jnp.float32),
    )(h_i, h_j, ea, w1a, w1b, w1c, b1.reshape(1, HIDDEN), w2,
      b2.reshape(1, NODE_DIM))


def kernel(atoms, edge_index, coordinates, is_receptor, batch, node_table,
           subunit_table, W1s, b1s, W2s, b2s, fc_w, fc_b):
    src = edge_index[0]
    dst = edge_index[1]
    h = jnp.take(node_table, atoms, axis=0)
    diff = coordinates[src] - coordinates[dst]
    dist = jnp.sqrt(jnp.sum(diff * diff, axis=1) + 1e-12)
    offset = jnp.linspace(0.0, 5.0, EDGE_DIM // 2)
    coeff = -0.5 / (offset[1] - offset[0]) ** 2
    rbf = jnp.exp(coeff * jnp.square(dist[:, None] - offset[None, :]))
    is_inter = (is_receptor[src] != is_receptor[dst]).astype(jnp.int32)
    edge_kind = jnp.take(subunit_table, is_inter, axis=0)
    edge_attr = jnp.concatenate([rbf, edge_kind], axis=1)
    dst3 = dst.reshape(_NW, _SCI, _SCC)
    zeros_n = jnp.zeros((_NP, NODE_DIM), jnp.float32)
    for i in range(N_LAYERS):
        h_j, h_i = _sc_gather2(h, src, dst)
        m = _mlp(h_i, h_j, edge_attr, W1s[i], b1s[i], W2s[i], b2s[i])
        acc = _sc_scatter_add(m, dst3, zeros_n)
        h = jax.nn.relu(h + acc[0, :N] + acc[1, :N])
    sums = jax.ops.segment_sum(h, batch, num_segments=N_GRAPHS)
    counts = jax.ops.segment_sum(jnp.ones((N,), dtype=h.dtype), batch,
                                 num_segments=N_GRAPHS)
    pooled = sums / jnp.clip(counts, 1.0)[:, None]
    out = pooled @ fc_w + fc_b
    return out.squeeze(-1)


# speedup vs baseline: 5.2796x; 3.3742x over previous
import functools

import jax
import jax.numpy as jnp
from jax import lax
from jax.experimental import pallas as pl
from jax.experimental.pallas import tpu as pltpu
from jax.experimental.pallas import tpu_sc as plsc

N = 10000
E = 320000
NODE_DIM = 128
EDGE_DIM = 16
HIDDEN = 2 * NODE_DIM
N_LAYERS = 3
N_GRAPHS = 16

EBLK = 2000

_NC = 2
_NS = 16
_NW = _NC * _NS
_EPW = E // _NW
_GC = 400

_sc_mesh = lambda: plsc.VectorSubcoreMesh(core_axis_name="c",
                                          subcore_axis_name="s")


@functools.cache
def _make_gather2(n_rows, width, chunk):
    per_w = n_rows // _NW

    @functools.partial(
        pl.kernel,
        out_type=(jax.ShapeDtypeStruct((n_rows, width), jnp.float32),
                  jax.ShapeDtypeStruct((n_rows, width), jnp.float32)),
        mesh=_sc_mesh(),
        scratch_types=[
            pltpu.VMEM((chunk,), jnp.int32),
            pltpu.VMEM((chunk, width), jnp.float32),
            pltpu.VMEM((chunk,), jnp.int32),
            pltpu.VMEM((chunk, width), jnp.float32),
            pltpu.SemaphoreType.DMA,
            pltpu.SemaphoreType.DMA,
        ],
    )
    def gather2(tab_hbm, src_hbm, dst_hbm, oj_hbm, oi_hbm,
                sidx, srows, didx, drows, sem_s, sem_d):
        wid = lax.axis_index("s") * _NC + lax.axis_index("c")
        base0 = wid * per_w

        def body(it, carry):
            base = base0 + it * chunk
            pltpu.sync_copy(src_hbm.at[pl.ds(base, chunk)], sidx)
            pltpu.sync_copy(dst_hbm.at[pl.ds(base, chunk)], didx)
            cp_s = pltpu.async_copy(tab_hbm.at[sidx], srows, sem_s)
            cp_d = pltpu.async_copy(tab_hbm.at[didx], drows, sem_d)
            cp_s.wait()
            cp_d.wait()
            pltpu.sync_copy(srows, oj_hbm.at[pl.ds(base, chunk)])
            pltpu.sync_copy(drows, oi_hbm.at[pl.ds(base, chunk)])
            return carry

        lax.fori_loop(0, per_w // chunk, body, 0)

    return gather2


_GEOC = 400


@functools.partial(
    pl.kernel,
    out_type=(jax.ShapeDtypeStruct((E,), jnp.float32),
              jax.ShapeDtypeStruct((E,), jnp.float32)),
    mesh=_sc_mesh(),
    scratch_types=[
        pltpu.VMEM((4 * N,), jnp.float32),
        pltpu.VMEM((_GEOC,), jnp.int32),
        pltpu.VMEM((_GEOC,), jnp.int32),
        pltpu.VMEM((_GEOC,), jnp.float32),
        pltpu.VMEM((_GEOC,), jnp.float32),
    ],
    compiler_params=pltpu.CompilerParams(needs_layout_passes=False),
)
def _sc_edge_geo(geo_hbm, src_hbm, dst_hbm, d2_hbm, t_hbm,
                 geo_v, sidx, didx, d2_v, t_v):
    wid = lax.axis_index("s") * _NC + lax.axis_index("c")
    base0 = wid * _EPW
    pltpu.sync_copy(geo_hbm, geo_v)

    def chunk_body(it, carry):
        base = base0 + it * _GEOC
        pltpu.sync_copy(src_hbm.at[pl.ds(base, _GEOC)], sidx)
        pltpu.sync_copy(dst_hbm.at[pl.ds(base, _GEOC)], didx)

        def grp_body(g, carry2):
            sl = pl.ds(g * 16, 16)
            sv = sidx[sl]
            dv = didx[sl]
            dx = (plsc.load_gather(geo_v, [sv])
                  - plsc.load_gather(geo_v, [dv]))
            dy = (plsc.load_gather(geo_v, [sv + N])
                  - plsc.load_gather(geo_v, [dv + N]))
            dz = (plsc.load_gather(geo_v, [sv + 2 * N])
                  - plsc.load_gather(geo_v, [dv + 2 * N]))
            rs = plsc.load_gather(geo_v, [sv + 3 * N])
            rd = plsc.load_gather(geo_v, [dv + 3 * N])
            d2_v[sl] = dx * dx + dy * dy + dz * dz
            t_v[sl] = jnp.where(rs != rd, 1.0, 0.0).astype(jnp.float32)
            return carry2

        lax.fori_loop(0, _GEOC // 16, grp_body, 0)
        pltpu.sync_copy(d2_v, d2_hbm.at[pl.ds(base, _GEOC)])
        pltpu.sync_copy(t_v, t_hbm.at[pl.ds(base, _GEOC)])
        return carry

    lax.fori_loop(0, _EPW // _GEOC, chunk_body, 0)


@functools.cache
def _make_gather1(n_rows, width, chunk):
    per_w = n_rows // _NW

    @functools.partial(
        pl.kernel,
        out_type=jax.ShapeDtypeStruct((n_rows, width), jnp.float32),
        mesh=_sc_mesh(),
        scratch_types=[
            pltpu.VMEM((chunk,), jnp.int32),
            pltpu.VMEM((chunk, width), jnp.float32),
            pltpu.SemaphoreType.DMA,
        ],
    )
    def gather1(tab_hbm, idx_hbm, out_hbm, vidx, vrows, sem):
        wid = lax.axis_index("s") * _NC + lax.axis_index("c")
        base0 = wid * per_w

        def body(it, carry):
            base = base0 + it * chunk
            pltpu.sync_copy(idx_hbm.at[pl.ds(base, chunk)], vidx)
            pltpu.async_copy(tab_hbm.at[vidx], vrows, sem).wait()
            pltpu.sync_copy(vrows, out_hbm.at[pl.ds(base, chunk)])
            return carry

        lax.fori_loop(0, per_w // chunk, body, 0)

    return gather1


_SCC = 80
_SCI = _EPW // _SCC
_NP = 10240
_NPT = _NP // _NS


@functools.partial(
    pl.kernel,
    out_type=jax.ShapeDtypeStruct((_NC, _NP, NODE_DIM), jnp.float32),
    mesh=_sc_mesh(),
    scratch_types=[
        pltpu.VMEM((_SCI, _SCC), jnp.int32),
        pltpu.VMEM((_SCC, NODE_DIM), jnp.float32),
        pltpu.VMEM_SHARED((_NP, NODE_DIM), jnp.float32),
    ],
)
def _sc_scatter_add(m_hbm, dst3_hbm, zeros_hbm, out_hbm, idx_v, rows_v, acc_sh):
    cid = lax.axis_index("c")
    sid = lax.axis_index("s")
    wid = sid * _NC + cid
    base0 = wid * _EPW
    pltpu.sync_copy(zeros_hbm.at[pl.ds(sid * _NPT, _NPT)],
                    acc_sh.at[pl.ds(sid * _NPT, _NPT)])
    pltpu.sync_copy(dst3_hbm.at[wid], idx_v)
    plsc.subcore_barrier()

    def body(j, carry):
        pltpu.sync_copy(m_hbm.at[pl.ds(base0 + j * _SCC, _SCC)], rows_v)
        pltpu.sync_copy(rows_v, acc_sh.at[idx_v.at[j]], add=True)
        return carry

    lax.fori_loop(0, _SCI, body, 0)
    plsc.subcore_barrier()
    pltpu.sync_copy(acc_sh.at[pl.ds(sid * _NPT, _NPT)],
                    out_hbm.at[cid, pl.ds(sid * _NPT, _NPT)])


def _mlp_body(hi_ref, hj_ref, ea_ref, w1a_ref, w1b_ref, w1c_ref, b1_ref,
              w2_ref, b2_ref, out_ref):
    acc = jnp.dot(hi_ref[...], w1a_ref[...], preferred_element_type=jnp.float32)
    acc += jnp.dot(hj_ref[...], w1b_ref[...], preferred_element_type=jnp.float32)
    acc += jnp.dot(ea_ref[...], w1c_ref[...], preferred_element_type=jnp.float32)
    acc += b1_ref[...]
    s = acc * jax.nn.sigmoid(acc)
    out = jnp.dot(s, w2_ref[...], preferred_element_type=jnp.float32)
    out_ref[...] = out + b2_ref[...]


@jax.jit
def _mlp(h_i, h_j, ea, w1, b1, w2, b2):
    w1a = w1[:NODE_DIM]
    w1b = w1[NODE_DIM:2 * NODE_DIM]
    w1c = w1[2 * NODE_DIM:]
    grid = (E // EBLK,)
    full = lambda shape: pl.BlockSpec(shape, lambda i: (0,) * len(shape))
    return pl.pallas_call(
        _mlp_body,
        grid=grid,
        in_specs=[
            pl.BlockSpec((EBLK, NODE_DIM), lambda i: (i, 0)),
            pl.BlockSpec((EBLK, NODE_DIM), lambda i: (i, 0)),
            pl.BlockSpec((EBLK, EDGE_DIM), lambda i: (i, 0)),
            full((NODE_DIM, HIDDEN)),
            full((NODE_DIM, HIDDEN)),
            full((EDGE_DIM, HIDDEN)),
            full((1, HIDDEN)),
            full((HIDDEN, NODE_DIM)),
            full((1, NODE_DIM)),
        ],
        out_specs=pl.BlockSpec((EBLK, NODE_DIM), lambda i: (i, 0)),
        out_shape=jax.ShapeDtypeStruct((E, NODE_DIM), jnp.float32),
    )(h_i, h_j, ea, w1a, w1b, w1c, b1.reshape(1, HIDDEN), w2,
      b2.reshape(1, NODE_DIM))


_RBF_STEP = 5.0 / 7.0
_RBF_COEFF = -0.5 / _RBF_STEP ** 2


def _ea_body(d2_ref, t_ref, st_ref, out_ref):
    dist = jnp.sqrt(d2_ref[...] + 1e-12)
    off = (jax.lax.broadcasted_iota(jnp.int32, (1, EDGE_DIM // 2), 1)
           .astype(jnp.float32) * _RBF_STEP)
    rbf = jnp.exp(_RBF_COEFF * jnp.square(dist - off))
    t = t_ref[...]
    s0 = st_ref[0:1, :]
    s1 = st_ref[1:2, :]
    kind = s0 + t * (s1 - s0)
    out_ref[...] = jnp.concatenate([rbf, kind], axis=1)


@jax.jit
def _edge_attr(d2, t, st_pad):
    return pl.pallas_call(
        _ea_body,
        grid=(E // EBLK,),
        in_specs=[
            pl.BlockSpec((EBLK, 1), lambda i: (i, 0)),
            pl.BlockSpec((EBLK, 1), lambda i: (i, 0)),
            pl.BlockSpec((8, EDGE_DIM // 2), lambda i: (0, 0)),
        ],
        out_specs=pl.BlockSpec((EBLK, EDGE_DIM), lambda i: (i, 0)),
        out_shape=jax.ShapeDtypeStruct((E, EDGE_DIM), jnp.float32),
    )(d2, t, st_pad)


_PBLK = 2000


def _pool_body(b_ref, h_ref, fcw_ref, fcb_ref, out_ref, sums, cnts):
    i = pl.program_id(0)

    @pl.when(i == 0)
    def _():
        sums[...] = jnp.zeros_like(sums)
        cnts[...] = jnp.zeros_like(cnts)

    b = b_ref[pl.ds(i * _PBLK, _PBLK), :]
    g = jax.lax.broadcasted_iota(jnp.int32, (1, N_GRAPHS), 1).astype(jnp.float32)
    s = (b == g).astype(jnp.float32)
    dims = (((0,), (0,)), ((), ()))
    sums[...] += jax.lax.dot_general(s, h_ref[...], dims,
                                     preferred_element_type=jnp.float32)
    cnts[...] += jax.lax.dot_general(
        s, jnp.ones_like(h_ref[...]), dims,
        preferred_element_type=jnp.float32)

    @pl.when(i == (N // _PBLK) - 1)
    def _():
        pooled = sums[...] / jnp.maximum(cnts[...], 1.0)
        out_ref[...] = jnp.dot(pooled, fcw_ref[...],
                               preferred_element_type=jnp.float32) + fcb_ref[...]


@jax.jit
def _pool_fc(batch_f, h, fc_w, fc_b):
    return pl.pallas_call(
        _pool_body,
        grid=(N // _PBLK,),
        in_specs=[
            pl.BlockSpec((N, 1), lambda i: (0, 0)),
            pl.BlockSpec((_PBLK, NODE_DIM), lambda i: (i, 0)),
            pl.BlockSpec((NODE_DIM, 1), lambda i: (0, 0)),
            pl.BlockSpec((1, 1), lambda i: (0, 0)),
        ],
        out_specs=pl.BlockSpec((N_GRAPHS, 1), lambda i: (0, 0)),
        out_shape=jax.ShapeDtypeStruct((N_GRAPHS, 1), jnp.float32),
        scratch_shapes=[
            pltpu.VMEM((N_GRAPHS, NODE_DIM), jnp.float32),
            pltpu.VMEM((N_GRAPHS, NODE_DIM), jnp.float32),
        ],
    )(batch_f, h, fc_w, fc_b)


_NPAD = 10240


def kernel(atoms, edge_index, coordinates, is_receptor, batch, node_table,
           subunit_table, W1s, b1s, W2s, b2s, fc_w, fc_b):
    src = edge_index[0]
    dst = edge_index[1]
    atoms_p = jnp.concatenate(
        [atoms, jnp.zeros((_NPAD - N,), atoms.dtype)]).astype(jnp.int32)
    h = _make_gather1(_NPAD, NODE_DIM, 320)(node_table, atoms_p)[:N]
    geo = jnp.concatenate(
        [coordinates.T.reshape(-1),
         is_receptor.astype(jnp.float32)])
    d2, t = _sc_edge_geo(geo, src, dst)
    st_pad = jnp.zeros((8, EDGE_DIM // 2), jnp.float32).at[:2].set(subunit_table)
    edge_attr = _edge_attr(d2[:, None], t[:, None], st_pad)
    dst3 = dst.reshape(_NW, _SCI, _SCC)
    zeros_n = jnp.zeros((_NP, NODE_DIM), jnp.float32)
    gather_h = _make_gather2(E, NODE_DIM, _GC)
    for i in range(N_LAYERS):
        h_j, h_i = gather_h(h, src, dst)
        m = _mlp(h_i, h_j, edge_attr, W1s[i], b1s[i], W2s[i], b2s[i])
        acc = _sc_scatter_add(m, dst3, zeros_n)
        h = jax.nn.relu(h + acc[0, :N] + acc[1, :N])
    out = _pool_fc(batch.astype(jnp.float32)[:, None], h, fc_w,
                   fc_b.reshape(1, 1))
    return out[:, 0]


# speedup vs baseline: 5.3558x; 1.0144x over previous
import functools

import jax
import jax.numpy as jnp
from jax import lax
from jax.experimental import pallas as pl
from jax.experimental.pallas import tpu as pltpu
from jax.experimental.pallas import tpu_sc as plsc

N = 10000
E = 320000
NODE_DIM = 128
EDGE_DIM = 16
HIDDEN = 2 * NODE_DIM
N_LAYERS = 3
N_GRAPHS = 16

EBLK = 2000

_NC = 2
_NS = 16
_NW = _NC * _NS
_EPW = E // _NW
_GC = 400

_sc_mesh = lambda: plsc.VectorSubcoreMesh(core_axis_name="c",
                                          subcore_axis_name="s")


@functools.cache
def _make_gather2(n_rows, width, chunk):
    per_w = n_rows // _NW
    n_it = per_w // chunk
    assert n_it % 2 == 0

    @functools.partial(
        pl.kernel,
        out_type=(jax.ShapeDtypeStruct((n_rows, width), jnp.float32),
                  jax.ShapeDtypeStruct((n_rows, width), jnp.float32)),
        mesh=_sc_mesh(),
        scratch_types=[
            pltpu.VMEM((chunk,), jnp.int32),
            pltpu.VMEM((chunk, width), jnp.float32),
            pltpu.VMEM((chunk,), jnp.int32),
            pltpu.VMEM((chunk, width), jnp.float32),
            pltpu.VMEM((chunk,), jnp.int32),
            pltpu.VMEM((chunk, width), jnp.float32),
            pltpu.VMEM((chunk,), jnp.int32),
            pltpu.VMEM((chunk, width), jnp.float32),
            pltpu.SemaphoreType.DMA,
            pltpu.SemaphoreType.DMA,
            pltpu.SemaphoreType.DMA,
            pltpu.SemaphoreType.DMA,
            pltpu.SemaphoreType.DMA,
            pltpu.SemaphoreType.DMA,
            pltpu.SemaphoreType.DMA,
            pltpu.SemaphoreType.DMA,
        ],
    )
    def gather2(tab_hbm, src_hbm, dst_hbm, oj_hbm, oi_hbm,
                sidx_a, srows_a, didx_a, drows_a,
                sidx_b, srows_b, didx_b, drows_b,
                sem_sa, sem_da, sem_sb, sem_db,
                sem_wja, sem_wia, sem_wjb, sem_wib):
        wid = lax.axis_index("s") * _NC + lax.axis_index("c")
        base0 = wid * per_w

        def wait_writes(srows, drows, sem_wj, sem_wi):
            pltpu.make_async_copy(srows, oj_hbm.at[pl.ds(base0, chunk)],
                                  sem_wj).wait()
            pltpu.make_async_copy(drows, oi_hbm.at[pl.ds(base0, chunk)],
                                  sem_wi).wait()

        def process(c, first, sidx, didx, srows, drows, sem_s, sem_d,
                    sem_wj, sem_wi):
            base = base0 + c * chunk

            @pl.when(jnp.logical_not(first))
            def _():
                wait_writes(srows, drows, sem_wj, sem_wi)

            pltpu.sync_copy(src_hbm.at[pl.ds(base, chunk)], sidx)
            pltpu.sync_copy(dst_hbm.at[pl.ds(base, chunk)], didx)
            cp_s = pltpu.async_copy(tab_hbm.at[sidx], srows, sem_s)
            cp_d = pltpu.async_copy(tab_hbm.at[didx], drows, sem_d)
            cp_s.wait()
            cp_d.wait()
            pltpu.async_copy(srows, oj_hbm.at[pl.ds(base, chunk)], sem_wj)
            pltpu.async_copy(drows, oi_hbm.at[pl.ds(base, chunk)], sem_wi)

        def body(k, carry):
            process(2 * k, k == 0, sidx_a, didx_a, srows_a, drows_a,
                    sem_sa, sem_da, sem_wja, sem_wia)
            process(2 * k + 1, k == 0, sidx_b, didx_b, srows_b, drows_b,
                    sem_sb, sem_db, sem_wjb, sem_wib)
            return carry

        lax.fori_loop(0, n_it // 2, body, 0)
        wait_writes(srows_a, drows_a, sem_wja, sem_wia)
        wait_writes(srows_b, drows_b, sem_wjb, sem_wib)

    return gather2


_GEOC = 400


@functools.partial(
    pl.kernel,
    out_type=(jax.ShapeDtypeStruct((E,), jnp.float32),
              jax.ShapeDtypeStruct((E,), jnp.float32)),
    mesh=_sc_mesh(),
    scratch_types=[
        pltpu.VMEM((4 * N,), jnp.float32),
        pltpu.VMEM((_GEOC,), jnp.int32),
        pltpu.VMEM((_GEOC,), jnp.int32),
        pltpu.VMEM((_GEOC,), jnp.float32),
        pltpu.VMEM((_GEOC,), jnp.float32),
    ],
    compiler_params=pltpu.CompilerParams(needs_layout_passes=False),
)
def _sc_edge_geo(geo_hbm, src_hbm, dst_hbm, d2_hbm, t_hbm,
                 geo_v, sidx, didx, d2_v, t_v):
    wid = lax.axis_index("s") * _NC + lax.axis_index("c")
    base0 = wid * _EPW
    pltpu.sync_copy(geo_hbm, geo_v)

    def chunk_body(it, carry):
        base = base0 + it * _GEOC
        pltpu.sync_copy(src_hbm.at[pl.ds(base, _GEOC)], sidx)
        pltpu.sync_copy(dst_hbm.at[pl.ds(base, _GEOC)], didx)

        def grp_body(g, carry2):
            sl = pl.ds(g * 16, 16)
            sv = sidx[sl]
            dv = didx[sl]
            dx = (plsc.load_gather(geo_v, [sv])
                  - plsc.load_gather(geo_v, [dv]))
            dy = (plsc.load_gather(geo_v, [sv + N])
                  - plsc.load_gather(geo_v, [dv + N]))
            dz = (plsc.load_gather(geo_v, [sv + 2 * N])
                  - plsc.load_gather(geo_v, [dv + 2 * N]))
            rs = plsc.load_gather(geo_v, [sv + 3 * N])
            rd = plsc.load_gather(geo_v, [dv + 3 * N])
            d2_v[sl] = dx * dx + dy * dy + dz * dz
            t_v[sl] = jnp.where(rs != rd, 1.0, 0.0).astype(jnp.float32)
            return carry2

        lax.fori_loop(0, _GEOC // 16, grp_body, 0)
        pltpu.sync_copy(d2_v, d2_hbm.at[pl.ds(base, _GEOC)])
        pltpu.sync_copy(t_v, t_hbm.at[pl.ds(base, _GEOC)])
        return carry

    lax.fori_loop(0, _EPW // _GEOC, chunk_body, 0)


@functools.cache
def _make_gather1(n_rows, width, chunk):
    per_w = n_rows // _NW

    @functools.partial(
        pl.kernel,
        out_type=jax.ShapeDtypeStruct((n_rows, width), jnp.float32),
        mesh=_sc_mesh(),
        scratch_types=[
            pltpu.VMEM((chunk,), jnp.int32),
            pltpu.VMEM((chunk, width), jnp.float32),
            pltpu.SemaphoreType.DMA,
        ],
    )
    def gather1(tab_hbm, idx_hbm, out_hbm, vidx, vrows, sem):
        wid = lax.axis_index("s") * _NC + lax.axis_index("c")
        base0 = wid * per_w

        def body(it, carry):
            base = base0 + it * chunk
            pltpu.sync_copy(idx_hbm.at[pl.ds(base, chunk)], vidx)
            pltpu.async_copy(tab_hbm.at[vidx], vrows, sem).wait()
            pltpu.sync_copy(vrows, out_hbm.at[pl.ds(base, chunk)])
            return carry

        lax.fori_loop(0, per_w // chunk, body, 0)

    return gather1


_SCC = 80
_SCI = _EPW // _SCC
_NP = 10240
_NPT = _NP // _NS


@functools.partial(
    pl.kernel,
    out_type=jax.ShapeDtypeStruct((_NC, _NP, NODE_DIM), jnp.float32),
    mesh=_sc_mesh(),
    scratch_types=[
        pltpu.VMEM((_SCI, _SCC), jnp.int32),
        pltpu.VMEM((_SCC, NODE_DIM), jnp.float32),
        pltpu.VMEM_SHARED((_NP, NODE_DIM), jnp.float32),
    ],
)
def _sc_scatter_add(m_hbm, dst3_hbm, zeros_hbm, out_hbm, idx_v, rows_v, acc_sh):
    cid = lax.axis_index("c")
    sid = lax.axis_index("s")
    wid = sid * _NC + cid
    base0 = wid * _EPW
    pltpu.sync_copy(zeros_hbm.at[pl.ds(sid * _NPT, _NPT)],
                    acc_sh.at[pl.ds(sid * _NPT, _NPT)])
    pltpu.sync_copy(dst3_hbm.at[wid], idx_v)
    plsc.subcore_barrier()

    def body(j, carry):
        pltpu.sync_copy(m_hbm.at[pl.ds(base0 + j * _SCC, _SCC)], rows_v)
        pltpu.sync_copy(rows_v, acc_sh.at[idx_v.at[j]], add=True)
        return carry

    lax.fori_loop(0, _SCI, body, 0)
    plsc.subcore_barrier()
    pltpu.sync_copy(acc_sh.at[pl.ds(sid * _NPT, _NPT)],
                    out_hbm.at[cid, pl.ds(sid * _NPT, _NPT)])


def _mlp_body(hi_ref, hj_ref, ea_ref, w1a_ref, w1b_ref, w1c_ref, b1_ref,
              w2_ref, b2_ref, out_ref):
    acc = jnp.dot(hi_ref[...], w1a_ref[...], preferred_element_type=jnp.float32)
    acc += jnp.dot(hj_ref[...], w1b_ref[...], preferred_element_type=jnp.float32)
    acc += jnp.dot(ea_ref[...], w1c_ref[...], preferred_element_type=jnp.float32)
    acc += b1_ref[...]
    s = acc * jax.nn.sigmoid(acc)
    out = jnp.dot(s, w2_ref[...], preferred_element_type=jnp.float32)
    out_ref[...] = out + b2_ref[...]


@jax.jit
def _mlp(h_i, h_j, ea, w1, b1, w2, b2):
    w1a = w1[:NODE_DIM]
    w1b = w1[NODE_DIM:2 * NODE_DIM]
    w1c = w1[2 * NODE_DIM:]
    grid = (E // EBLK,)
    full = lambda shape: pl.BlockSpec(shape, lambda i: (0,) * len(shape))
    return pl.pallas_call(
        _mlp_body,
        grid=grid,
        in_specs=[
            pl.BlockSpec((EBLK, NODE_DIM), lambda i: (i, 0)),
            pl.BlockSpec((EBLK, NODE_DIM), lambda i: (i, 0)),
            pl.BlockSpec((EBLK, EDGE_DIM), lambda i: (i, 0)),
            full((NODE_DIM, HIDDEN)),
            full((NODE_DIM, HIDDEN)),
            full((EDGE_DIM, HIDDEN)),
            full((1, HIDDEN)),
            full((HIDDEN, NODE_DIM)),
            full((1, NODE_DIM)),
        ],
        out_specs=pl.BlockSpec((EBLK, NODE_DIM), lambda i: (i, 0)),
        out_shape=jax.ShapeDtypeStruct((E, NODE_DIM), jnp.float32),
    )(h_i, h_j, ea, w1a, w1b, w1c, b1.reshape(1, HIDDEN), w2,
      b2.reshape(1, NODE_DIM))


_RBF_STEP = 5.0 / 7.0
_RBF_COEFF = -0.5 / _RBF_STEP ** 2


def _ea_body(d2_ref, t_ref, st_ref, out_ref):
    dist = jnp.sqrt(d2_ref[...] + 1e-12)
    off = (jax.lax.broadcasted_iota(jnp.int32, (1, EDGE_DIM // 2), 1)
           .astype(jnp.float32) * _RBF_STEP)
    rbf = jnp.exp(_RBF_COEFF * jnp.square(dist - off))
    t = t_ref[...]
    s0 = st_ref[0:1, :]
    s1 = st_ref[1:2, :]
    kind = s0 + t * (s1 - s0)
    out_ref[...] = jnp.concatenate([rbf, kind], axis=1)


@jax.jit
def _edge_attr(d2, t, st_pad):
    return pl.pallas_call(
        _ea_body,
        grid=(E // EBLK,),
        in_specs=[
            pl.BlockSpec((EBLK, 1), lambda i: (i, 0)),
            pl.BlockSpec((EBLK, 1), lambda i: (i, 0)),
            pl.BlockSpec((8, EDGE_DIM // 2), lambda i: (0, 0)),
        ],
        out_specs=pl.BlockSpec((EBLK, EDGE_DIM), lambda i: (i, 0)),
        out_shape=jax.ShapeDtypeStruct((E, EDGE_DIM), jnp.float32),
    )(d2, t, st_pad)


_PBLK = 2000


def _pool_body(b_ref, h_ref, fcw_ref, fcb_ref, out_ref, sums, cnts):
    i = pl.program_id(0)

    @pl.when(i == 0)
    def _():
        sums[...] = jnp.zeros_like(sums)
        cnts[...] = jnp.zeros_like(cnts)

    b = b_ref[pl.ds(i * _PBLK, _PBLK), :]
    g = jax.lax.broadcasted_iota(jnp.int32, (1, N_GRAPHS), 1).astype(jnp.float32)
    s = (b == g).astype(jnp.float32)
    dims = (((0,), (0,)), ((), ()))
    sums[...] += jax.lax.dot_general(s, h_ref[...], dims,
                                     preferred_element_type=jnp.float32)
    cnts[...] += jax.lax.dot_general(
        s, jnp.ones_like(h_ref[...]), dims,
        preferred_element_type=jnp.float32)

    @pl.when(i == (N // _PBLK) - 1)
    def _():
        pooled = sums[...] / jnp.maximum(cnts[...], 1.0)
        out_ref[...] = jnp.dot(pooled, fcw_ref[...],
                               preferred_element_type=jnp.float32) + fcb_ref[...]


@jax.jit
def _pool_fc(batch_f, h, fc_w, fc_b):
    return pl.pallas_call(
        _pool_body,
        grid=(N // _PBLK,),
        in_specs=[
            pl.BlockSpec((N, 1), lambda i: (0, 0)),
            pl.BlockSpec((_PBLK, NODE_DIM), lambda i: (i, 0)),
            pl.BlockSpec((NODE_DIM, 1), lambda i: (0, 0)),
            pl.BlockSpec((1, 1), lambda i: (0, 0)),
        ],
        out_specs=pl.BlockSpec((N_GRAPHS, 1), lambda i: (0, 0)),
        out_shape=jax.ShapeDtypeStruct((N_GRAPHS, 1), jnp.float32),
        scratch_shapes=[
            pltpu.VMEM((N_GRAPHS, NODE_DIM), jnp.float32),
            pltpu.VMEM((N_GRAPHS, NODE_DIM), jnp.float32),
        ],
    )(batch_f, h, fc_w, fc_b)


_NPAD = 10240


def kernel(atoms, edge_index, coordinates, is_receptor, batch, node_table,
           subunit_table, W1s, b1s, W2s, b2s, fc_w, fc_b):
    src = edge_index[0]
    dst = edge_index[1]
    atoms_p = jnp.concatenate(
        [atoms, jnp.zeros((_NPAD - N,), atoms.dtype)]).astype(jnp.int32)
    h = _make_gather1(_NPAD, NODE_DIM, 320)(node_table, atoms_p)[:N]
    geo = jnp.concatenate(
        [coordinates.T.reshape(-1),
         is_receptor.astype(jnp.float32)])
    d2, t = _sc_edge_geo(geo, src, dst)
    st_pad = jnp.zeros((8, EDGE_DIM // 2), jnp.float32).at[:2].set(subunit_table)
    edge_attr = _edge_attr(d2[:, None], t[:, None], st_pad)
    dst3 = dst.reshape(_NW, _SCI, _SCC)
    zeros_n = jnp.zeros((_NP, NODE_DIM), jnp.float32)
    gather_h = _make_gather2(E, NODE_DIM, 200)
    for i in range(N_LAYERS):
        h_j, h_i = gather_h(h, src, dst)
        m = _mlp(h_i, h_j, edge_attr, W1s[i], b1s[i], W2s[i], b2s[i])
        acc = _sc_scatter_add(m, dst3, zeros_n)
        h = jax.nn.relu(h + acc[0, :N] + acc[1, :N])
    out = _pool_fc(batch.astype(jnp.float32)[:, None], h, fc_w,
                   fc_b.reshape(1, 1))
    return out[:, 0]


# speedup vs baseline: 5.4023x; 1.0087x over previous
import functools

import jax
import jax.numpy as jnp
from jax import lax
from jax.experimental import pallas as pl
from jax.experimental.pallas import tpu as pltpu
from jax.experimental.pallas import tpu_sc as plsc

N = 10000
E = 320000
NODE_DIM = 128
EDGE_DIM = 16
HIDDEN = 2 * NODE_DIM
N_LAYERS = 3
N_GRAPHS = 16

EBLK = 2000

_NC = 2
_NS = 16
_NW = _NC * _NS
_EPW = E // _NW
_GC = 400

_sc_mesh = lambda: plsc.VectorSubcoreMesh(core_axis_name="c",
                                          subcore_axis_name="s")


@functools.cache
def _make_gather2(n_rows, width, chunk):
    per_w = n_rows // _NW
    n_it = per_w // chunk
    assert n_it % 2 == 0

    @functools.partial(
        pl.kernel,
        out_type=(jax.ShapeDtypeStruct((n_rows, width), jnp.float32),
                  jax.ShapeDtypeStruct((n_rows, width), jnp.float32)),
        mesh=_sc_mesh(),
        scratch_types=[
            pltpu.VMEM((chunk,), jnp.int32),
            pltpu.VMEM((chunk, width), jnp.float32),
            pltpu.VMEM((chunk,), jnp.int32),
            pltpu.VMEM((chunk, width), jnp.float32),
            pltpu.VMEM((chunk,), jnp.int32),
            pltpu.VMEM((chunk, width), jnp.float32),
            pltpu.VMEM((chunk,), jnp.int32),
            pltpu.VMEM((chunk, width), jnp.float32),
            pltpu.SemaphoreType.DMA,
            pltpu.SemaphoreType.DMA,
            pltpu.SemaphoreType.DMA,
            pltpu.SemaphoreType.DMA,
            pltpu.SemaphoreType.DMA,
            pltpu.SemaphoreType.DMA,
            pltpu.SemaphoreType.DMA,
            pltpu.SemaphoreType.DMA,
        ],
    )
    def gather2(tab_hbm, src_hbm, dst_hbm, oj_hbm, oi_hbm,
                sidx_a, srows_a, didx_a, drows_a,
                sidx_b, srows_b, didx_b, drows_b,
                sem_sa, sem_da, sem_sb, sem_db,
                sem_wja, sem_wia, sem_wjb, sem_wib):
        wid = lax.axis_index("s") * _NC + lax.axis_index("c")
        base0 = wid * per_w

        def wait_writes(srows, drows, sem_wj, sem_wi):
            pltpu.make_async_copy(srows, oj_hbm.at[pl.ds(base0, chunk)],
                                  sem_wj).wait()
            pltpu.make_async_copy(drows, oi_hbm.at[pl.ds(base0, chunk)],
                                  sem_wi).wait()

        def process(c, first, sidx, didx, srows, drows, sem_s, sem_d,
                    sem_wj, sem_wi):
            base = base0 + c * chunk

            @pl.when(jnp.logical_not(first))
            def _():
                wait_writes(srows, drows, sem_wj, sem_wi)

            pltpu.sync_copy(src_hbm.at[pl.ds(base, chunk)], sidx)
            pltpu.sync_copy(dst_hbm.at[pl.ds(base, chunk)], didx)
            cp_s = pltpu.async_copy(tab_hbm.at[sidx], srows, sem_s)
            cp_d = pltpu.async_copy(tab_hbm.at[didx], drows, sem_d)
            cp_s.wait()
            cp_d.wait()
            pltpu.async_copy(srows, oj_hbm.at[pl.ds(base, chunk)], sem_wj)
            pltpu.async_copy(drows, oi_hbm.at[pl.ds(base, chunk)], sem_wi)

        def body(k, carry):
            process(2 * k, k == 0, sidx_a, didx_a, srows_a, drows_a,
                    sem_sa, sem_da, sem_wja, sem_wia)
            process(2 * k + 1, k == 0, sidx_b, didx_b, srows_b, drows_b,
                    sem_sb, sem_db, sem_wjb, sem_wib)
            return carry

        lax.fori_loop(0, n_it // 2, body, 0)
        wait_writes(srows_a, drows_a, sem_wja, sem_wia)
        wait_writes(srows_b, drows_b, sem_wjb, sem_wib)

    return gather2


_GEOC = 2000


@functools.partial(
    pl.kernel,
    out_type=(jax.ShapeDtypeStruct((E,), jnp.float32),
              jax.ShapeDtypeStruct((E,), jnp.float32)),
    mesh=_sc_mesh(),
    scratch_types=[
        pltpu.VMEM((4 * N,), jnp.float32),
        pltpu.VMEM((_GEOC,), jnp.int32),
        pltpu.VMEM((_GEOC,), jnp.int32),
        pltpu.VMEM((_GEOC,), jnp.float32),
        pltpu.VMEM((_GEOC,), jnp.float32),
    ],
    compiler_params=pltpu.CompilerParams(needs_layout_passes=False),
)
def _sc_edge_geo(geo_hbm, src_hbm, dst_hbm, d2_hbm, t_hbm,
                 geo_v, sidx, didx, d2_v, t_v):
    wid = lax.axis_index("s") * _NC + lax.axis_index("c")
    base0 = wid * _EPW
    pltpu.sync_copy(geo_hbm, geo_v)

    def chunk_body(it, carry):
        base = base0 + it * _GEOC
        pltpu.sync_copy(src_hbm.at[pl.ds(base, _GEOC)], sidx)
        pltpu.sync_copy(dst_hbm.at[pl.ds(base, _GEOC)], didx)

        def grp_body(g, carry2):
            sl = pl.ds(g * 16, 16)
            sv = sidx[sl]
            dv = didx[sl]
            dx = (plsc.load_gather(geo_v, [sv])
                  - plsc.load_gather(geo_v, [dv]))
            dy = (plsc.load_gather(geo_v, [sv + N])
                  - plsc.load_gather(geo_v, [dv + N]))
            dz = (plsc.load_gather(geo_v, [sv + 2 * N])
                  - plsc.load_gather(geo_v, [dv + 2 * N]))
            rs = plsc.load_gather(geo_v, [sv + 3 * N])
            rd = plsc.load_gather(geo_v, [dv + 3 * N])
            d2_v[sl] = dx * dx + dy * dy + dz * dz
            t_v[sl] = jnp.where(rs != rd, 1.0, 0.0).astype(jnp.float32)
            return carry2

        lax.fori_loop(0, _GEOC // 16, grp_body, 0)
        pltpu.sync_copy(d2_v, d2_hbm.at[pl.ds(base, _GEOC)])
        pltpu.sync_copy(t_v, t_hbm.at[pl.ds(base, _GEOC)])
        return carry

    lax.fori_loop(0, _EPW // _GEOC, chunk_body, 0)


@functools.cache
def _make_gather1(n_rows, width, chunk):
    per_w = n_rows // _NW

    @functools.partial(
        pl.kernel,
        out_type=jax.ShapeDtypeStruct((n_rows, width), jnp.float32),
        mesh=_sc_mesh(),
        scratch_types=[
            pltpu.VMEM((chunk,), jnp.int32),
            pltpu.VMEM((chunk, width), jnp.float32),
            pltpu.SemaphoreType.DMA,
        ],
    )
    def gather1(tab_hbm, idx_hbm, out_hbm, vidx, vrows, sem):
        wid = lax.axis_index("s") * _NC + lax.axis_index("c")
        base0 = wid * per_w

        def body(it, carry):
            base = base0 + it * chunk
            pltpu.sync_copy(idx_hbm.at[pl.ds(base, chunk)], vidx)
            pltpu.async_copy(tab_hbm.at[vidx], vrows, sem).wait()
            pltpu.sync_copy(vrows, out_hbm.at[pl.ds(base, chunk)])
            return carry

        lax.fori_loop(0, per_w // chunk, body, 0)

    return gather1


_SCC = 80
_SCI = _EPW // _SCC
_NP = 10240
_NPT = _NP // _NS


@functools.partial(
    pl.kernel,
    out_type=jax.ShapeDtypeStruct((_NC, _NP, NODE_DIM), jnp.float32),
    mesh=_sc_mesh(),
    scratch_types=[
        pltpu.VMEM((_SCI, _SCC), jnp.int32),
        pltpu.VMEM((_SCC, NODE_DIM), jnp.float32),
        pltpu.VMEM_SHARED((_NP, NODE_DIM), jnp.float32),
    ],
)
def _sc_scatter_add(m_hbm, dst3_hbm, zeros_hbm, out_hbm, idx_v, rows_v, acc_sh):
    cid = lax.axis_index("c")
    sid = lax.axis_index("s")
    wid = sid * _NC + cid
    base0 = wid * _EPW
    pltpu.sync_copy(zeros_hbm.at[pl.ds(sid * _NPT, _NPT)],
                    acc_sh.at[pl.ds(sid * _NPT, _NPT)])
    pltpu.sync_copy(dst3_hbm.at[wid], idx_v)
    plsc.subcore_barrier()

    def body(j, carry):
        pltpu.sync_copy(m_hbm.at[pl.ds(base0 + j * _SCC, _SCC)], rows_v)
        pltpu.sync_copy(rows_v, acc_sh.at[idx_v.at[j]], add=True)
        return carry

    lax.fori_loop(0, _SCI, body, 0)
    plsc.subcore_barrier()
    pltpu.sync_copy(acc_sh.at[pl.ds(sid * _NPT, _NPT)],
                    out_hbm.at[cid, pl.ds(sid * _NPT, _NPT)])


def _mlp_body(hi_ref, hj_ref, ea_ref, w1a_ref, w1b_ref, w1c_ref, b1_ref,
              w2_ref, b2_ref, out_ref):
    acc = jnp.dot(hi_ref[...], w1a_ref[...], preferred_element_type=jnp.float32)
    acc += jnp.dot(hj_ref[...], w1b_ref[...], preferred_element_type=jnp.float32)
    acc += jnp.dot(ea_ref[...], w1c_ref[...], preferred_element_type=jnp.float32)
    acc += b1_ref[...]
    s = acc * jax.nn.sigmoid(acc)
    out = jnp.dot(s, w2_ref[...], preferred_element_type=jnp.float32)
    out_ref[...] = out + b2_ref[...]


@jax.jit
def _mlp(h_i, h_j, ea, w1, b1, w2, b2):
    w1a = w1[:NODE_DIM]
    w1b = w1[NODE_DIM:2 * NODE_DIM]
    w1c = w1[2 * NODE_DIM:]
    grid = (E // EBLK,)
    full = lambda shape: pl.BlockSpec(shape, lambda i: (0,) * len(shape))
    return pl.pallas_call(
        _mlp_body,
        grid=grid,
        in_specs=[
            pl.BlockSpec((EBLK, NODE_DIM), lambda i: (i, 0)),
            pl.BlockSpec((EBLK, NODE_DIM), lambda i: (i, 0)),
            pl.BlockSpec((EBLK, EDGE_DIM), lambda i: (i, 0)),
            full((NODE_DIM, HIDDEN)),
            full((NODE_DIM, HIDDEN)),
            full((EDGE_DIM, HIDDEN)),
            full((1, HIDDEN)),
            full((HIDDEN, NODE_DIM)),
            full((1, NODE_DIM)),
        ],
        out_specs=pl.BlockSpec((EBLK, NODE_DIM), lambda i: (i, 0)),
        out_shape=jax.ShapeDtypeStruct((E, NODE_DIM), jnp.float32),
    )(h_i, h_j, ea, w1a, w1b, w1c, b1.reshape(1, HIDDEN), w2,
      b2.reshape(1, NODE_DIM))


_RBF_STEP = 5.0 / 7.0
_RBF_COEFF = -0.5 / _RBF_STEP ** 2


def _ea_body(d2_ref, t_ref, st_ref, out_ref):
    dist = jnp.sqrt(d2_ref[...] + 1e-12)
    off = (jax.lax.broadcasted_iota(jnp.int32, (1, EDGE_DIM // 2), 1)
           .astype(jnp.float32) * _RBF_STEP)
    rbf = jnp.exp(_RBF_COEFF * jnp.square(dist - off))
    t = t_ref[...]
    s0 = st_ref[0:1, :]
    s1 = st_ref[1:2, :]
    kind = s0 + t * (s1 - s0)
    out_ref[...] = jnp.concatenate([rbf, kind], axis=1)


@jax.jit
def _edge_attr(d2, t, st_pad):
    return pl.pallas_call(
        _ea_body,
        grid=(E // EBLK,),
        in_specs=[
            pl.BlockSpec((EBLK, 1), lambda i: (i, 0)),
            pl.BlockSpec((EBLK, 1), lambda i: (i, 0)),
            pl.BlockSpec((8, EDGE_DIM // 2), lambda i: (0, 0)),
        ],
        out_specs=pl.BlockSpec((EBLK, EDGE_DIM), lambda i: (i, 0)),
        out_shape=jax.ShapeDtypeStruct((E, EDGE_DIM), jnp.float32),
    )(d2, t, st_pad)


_PBLK = 2000


def _pool_body(b_ref, h_ref, fcw_ref, fcb_ref, out_ref, sums, cnts):
    i = pl.program_id(0)

    @pl.when(i == 0)
    def _():
        sums[...] = jnp.zeros_like(sums)
        cnts[...] = jnp.zeros_like(cnts)

    b = b_ref[pl.ds(i * _PBLK, _PBLK), :]
    g = jax.lax.broadcasted_iota(jnp.int32, (1, N_GRAPHS), 1).astype(jnp.float32)
    s = (b == g).astype(jnp.float32)
    dims = (((0,), (0,)), ((), ()))
    sums[...] += jax.lax.dot_general(s, h_ref[...], dims,
                                     preferred_element_type=jnp.float32)
    cnts[...] += jax.lax.dot_general(
        s, jnp.ones_like(h_ref[...]), dims,
        preferred_element_type=jnp.float32)

    @pl.when(i == (N // _PBLK) - 1)
    def _():
        pooled = sums[...] / jnp.maximum(cnts[...], 1.0)
        out_ref[...] = jnp.dot(pooled, fcw_ref[...],
                               preferred_element_type=jnp.float32) + fcb_ref[...]


@jax.jit
def _pool_fc(batch_f, h, fc_w, fc_b):
    return pl.pallas_call(
        _pool_body,
        grid=(N // _PBLK,),
        in_specs=[
            pl.BlockSpec((N, 1), lambda i: (0, 0)),
            pl.BlockSpec((_PBLK, NODE_DIM), lambda i: (i, 0)),
            pl.BlockSpec((NODE_DIM, 1), lambda i: (0, 0)),
            pl.BlockSpec((1, 1), lambda i: (0, 0)),
        ],
        out_specs=pl.BlockSpec((N_GRAPHS, 1), lambda i: (0, 0)),
        out_shape=jax.ShapeDtypeStruct((N_GRAPHS, 1), jnp.float32),
        scratch_shapes=[
            pltpu.VMEM((N_GRAPHS, NODE_DIM), jnp.float32),
            pltpu.VMEM((N_GRAPHS, NODE_DIM), jnp.float32),
        ],
    )(batch_f, h, fc_w, fc_b)


_NPAD = 10240


def kernel(atoms, edge_index, coordinates, is_receptor, batch, node_table,
           subunit_table, W1s, b1s, W2s, b2s, fc_w, fc_b):
    src = edge_index[0]
    dst = edge_index[1]
    atoms_p = jnp.concatenate(
        [atoms, jnp.zeros((_NPAD - N,), atoms.dtype)]).astype(jnp.int32)
    h = _make_gather1(_NPAD, NODE_DIM, 320)(node_table, atoms_p)[:N]
    geo = jnp.concatenate(
        [coordinates.T.reshape(-1),
         is_receptor.astype(jnp.float32)])
    d2, t = _sc_edge_geo(geo, src, dst)
    st_pad = jnp.zeros((8, EDGE_DIM // 2), jnp.float32).at[:2].set(subunit_table)
    edge_attr = _edge_attr(d2[:, None], t[:, None], st_pad)
    dst3 = dst.reshape(_NW, _SCI, _SCC)
    zeros_n = jnp.zeros((_NP, NODE_DIM), jnp.float32)
    gather_h = _make_gather2(E, NODE_DIM, 200)
    for i in range(N_LAYERS):
        h_j, h_i = gather_h(h, src, dst)
        m = _mlp(h_i, h_j, edge_attr, W1s[i], b1s[i], W2s[i], b2s[i])
        acc = _sc_scatter_add(m, dst3, zeros_n)
        h = jax.nn.relu(h + acc[0, :N] + acc[1, :N])
    out = _pool_fc(batch.astype(jnp.float32)[:, None], h, fc_w,
                   fc_b.reshape(1, 1))
    return out[:, 0]


# speedup vs baseline: 5.9545x; 1.1022x over previous
import functools

import jax
import jax.numpy as jnp
from jax import lax
from jax.experimental import pallas as pl
from jax.experimental.pallas import tpu as pltpu
from jax.experimental.pallas import tpu_sc as plsc

N = 10000
E = 320000
NODE_DIM = 128
EDGE_DIM = 16
HIDDEN = 2 * NODE_DIM
N_LAYERS = 3
N_GRAPHS = 16

EBLK = 2000

_NC = 2
_NS = 16
_NW = _NC * _NS
_EPW = E // _NW
_GC = 400

_sc_mesh = lambda: plsc.VectorSubcoreMesh(core_axis_name="c",
                                          subcore_axis_name="s")


@functools.cache
def _make_gather2(n_rows, width, chunk, dtype=jnp.float32):
    per_w = n_rows // _NW
    n_it = per_w // chunk

    @functools.partial(
        pl.kernel,
        out_type=(jax.ShapeDtypeStruct((n_rows, width), dtype),
                  jax.ShapeDtypeStruct((n_rows, width), dtype)),
        mesh=_sc_mesh(),
        scratch_types=[
            pltpu.VMEM((chunk,), jnp.int32),
            pltpu.VMEM((chunk, width), dtype),
            pltpu.VMEM((chunk,), jnp.int32),
            pltpu.VMEM((chunk, width), dtype),
            pltpu.VMEM((chunk,), jnp.int32),
            pltpu.VMEM((chunk, width), dtype),
            pltpu.VMEM((chunk,), jnp.int32),
            pltpu.VMEM((chunk, width), dtype),
            pltpu.SemaphoreType.DMA,
            pltpu.SemaphoreType.DMA,
            pltpu.SemaphoreType.DMA,
            pltpu.SemaphoreType.DMA,
            pltpu.SemaphoreType.DMA,
            pltpu.SemaphoreType.DMA,
            pltpu.SemaphoreType.DMA,
            pltpu.SemaphoreType.DMA,
        ],
    )
    def gather2(tab_hbm, src_hbm, dst_hbm, oj_hbm, oi_hbm,
                sidx_a, srows_a, didx_a, drows_a,
                sidx_b, srows_b, didx_b, drows_b,
                sem_sa, sem_da, sem_sb, sem_db,
                sem_wja, sem_wia, sem_wjb, sem_wib):
        wid = lax.axis_index("s") * _NC + lax.axis_index("c")
        base0 = wid * per_w

        def wait_writes(srows, drows, sem_wj, sem_wi):
            pltpu.make_async_copy(srows, oj_hbm.at[pl.ds(base0, chunk)],
                                  sem_wj).wait()
            pltpu.make_async_copy(drows, oi_hbm.at[pl.ds(base0, chunk)],
                                  sem_wi).wait()

        def process(c, first, sidx, didx, srows, drows, sem_s, sem_d,
                    sem_wj, sem_wi):
            base = base0 + c * chunk

            @pl.when(jnp.logical_not(first))
            def _():
                wait_writes(srows, drows, sem_wj, sem_wi)

            pltpu.sync_copy(src_hbm.at[pl.ds(base, chunk)], sidx)
            pltpu.sync_copy(dst_hbm.at[pl.ds(base, chunk)], didx)
            cp_s = pltpu.async_copy(tab_hbm.at[sidx], srows, sem_s)
            cp_d = pltpu.async_copy(tab_hbm.at[didx], drows, sem_d)
            cp_s.wait()
            cp_d.wait()
            pltpu.async_copy(srows, oj_hbm.at[pl.ds(base, chunk)], sem_wj)
            pltpu.async_copy(drows, oi_hbm.at[pl.ds(base, chunk)], sem_wi)

        def body(k, carry):
            process(2 * k, k == 0, sidx_a, didx_a, srows_a, drows_a,
                    sem_sa, sem_da, sem_wja, sem_wia)
            process(2 * k + 1, k == 0, sidx_b, didx_b, srows_b, drows_b,
                    sem_sb, sem_db, sem_wjb, sem_wib)
            return carry

        lax.fori_loop(0, n_it // 2, body, 0)
        if n_it % 2:
            process(n_it - 1, n_it == 1, sidx_a, didx_a, srows_a, drows_a,
                    sem_sa, sem_da, sem_wja, sem_wia)
        wait_writes(srows_a, drows_a, sem_wja, sem_wia)
        if n_it > 1:
            wait_writes(srows_b, drows_b, sem_wjb, sem_wib)

    return gather2


_GEOC = 2000


@functools.partial(
    pl.kernel,
    out_type=(jax.ShapeDtypeStruct((E,), jnp.float32),
              jax.ShapeDtypeStruct((E,), jnp.float32)),
    mesh=_sc_mesh(),
    scratch_types=[
        pltpu.VMEM((4 * N,), jnp.float32),
        pltpu.VMEM((_GEOC,), jnp.int32),
        pltpu.VMEM((_GEOC,), jnp.int32),
        pltpu.VMEM((_GEOC,), jnp.float32),
        pltpu.VMEM((_GEOC,), jnp.float32),
    ],
    compiler_params=pltpu.CompilerParams(needs_layout_passes=False),
)
def _sc_edge_geo(geo_hbm, src_hbm, dst_hbm, d2_hbm, t_hbm,
                 geo_v, sidx, didx, d2_v, t_v):
    wid = lax.axis_index("s") * _NC + lax.axis_index("c")
    base0 = wid * _EPW
    pltpu.sync_copy(geo_hbm, geo_v)

    def chunk_body(it, carry):
        base = base0 + it * _GEOC
        pltpu.sync_copy(src_hbm.at[pl.ds(base, _GEOC)], sidx)
        pltpu.sync_copy(dst_hbm.at[pl.ds(base, _GEOC)], didx)

        def grp_body(g, carry2):
            sl = pl.ds(g * 16, 16)
            sv = sidx[sl]
            dv = didx[sl]
            dx = (plsc.load_gather(geo_v, [sv])
                  - plsc.load_gather(geo_v, [dv]))
            dy = (plsc.load_gather(geo_v, [sv + N])
                  - plsc.load_gather(geo_v, [dv + N]))
            dz = (plsc.load_gather(geo_v, [sv + 2 * N])
                  - plsc.load_gather(geo_v, [dv + 2 * N]))
            rs = plsc.load_gather(geo_v, [sv + 3 * N])
            rd = plsc.load_gather(geo_v, [dv + 3 * N])
            d2_v[sl] = dx * dx + dy * dy + dz * dz
            t_v[sl] = jnp.where(rs != rd, 1.0, 0.0).astype(jnp.float32)
            return carry2

        lax.fori_loop(0, _GEOC // 16, grp_body, 0)
        pltpu.sync_copy(d2_v, d2_hbm.at[pl.ds(base, _GEOC)])
        pltpu.sync_copy(t_v, t_hbm.at[pl.ds(base, _GEOC)])
        return carry

    lax.fori_loop(0, _EPW // _GEOC, chunk_body, 0)


@functools.cache
def _make_gather1(n_rows, width, chunk):
    per_w = n_rows // _NW

    @functools.partial(
        pl.kernel,
        out_type=jax.ShapeDtypeStruct((n_rows, width), jnp.float32),
        mesh=_sc_mesh(),
        scratch_types=[
            pltpu.VMEM((chunk,), jnp.int32),
            pltpu.VMEM((chunk, width), jnp.float32),
            pltpu.SemaphoreType.DMA,
        ],
    )
    def gather1(tab_hbm, idx_hbm, out_hbm, vidx, vrows, sem):
        wid = lax.axis_index("s") * _NC + lax.axis_index("c")
        base0 = wid * per_w

        def body(it, carry):
            base = base0 + it * chunk
            pltpu.sync_copy(idx_hbm.at[pl.ds(base, chunk)], vidx)
            pltpu.async_copy(tab_hbm.at[vidx], vrows, sem).wait()
            pltpu.sync_copy(vrows, out_hbm.at[pl.ds(base, chunk)])
            return carry

        lax.fori_loop(0, per_w // chunk, body, 0)

    return gather1


_SCC = 80
_SCI = _EPW // _SCC
_NP = 10240
_NPT = _NP // _NS


@functools.partial(
    pl.kernel,
    out_type=jax.ShapeDtypeStruct((_NC, _NP, NODE_DIM), jnp.float32),
    mesh=_sc_mesh(),
    scratch_types=[
        pltpu.VMEM((_SCI, _SCC), jnp.int32),
        pltpu.VMEM((_SCC, NODE_DIM), jnp.float32),
        pltpu.VMEM((_SCC, NODE_DIM), jnp.float32),
        pltpu.VMEM_SHARED((_NP, NODE_DIM), jnp.float32),
        pltpu.SemaphoreType.DMA,
        pltpu.SemaphoreType.DMA,
    ],
)
def _sc_scatter_add(m_hbm, dst3_hbm, zeros_hbm, out_hbm, idx_v, rows_v, rows_b,
                    acc_sh, sem_a, sem_b):
    cid = lax.axis_index("c")
    sid = lax.axis_index("s")
    wid = sid * _NC + cid
    base0 = wid * _EPW
    pltpu.sync_copy(zeros_hbm.at[pl.ds(sid * _NPT, _NPT)],
                    acc_sh.at[pl.ds(sid * _NPT, _NPT)])
    pltpu.sync_copy(dst3_hbm.at[wid], idx_v)
    plsc.subcore_barrier()

    def load(j, rows, sem):
        pltpu.async_copy(m_hbm.at[pl.ds(base0 + j * _SCC, _SCC)], rows, sem)

    def wait_load(j, rows, sem):
        pltpu.make_async_copy(m_hbm.at[pl.ds(base0 + j * _SCC, _SCC)],
                              rows, sem).wait()

    load(0, rows_v, sem_a)

    def body(k, carry):
        j0 = 2 * k
        j1 = j0 + 1
        load(j1, rows_b, sem_b)
        wait_load(j0, rows_v, sem_a)
        pltpu.sync_copy(rows_v, acc_sh.at[idx_v.at[j0]], add=True)

        @pl.when(j0 + 2 < _SCI)
        def _():
            load(j0 + 2, rows_v, sem_a)

        wait_load(j1, rows_b, sem_b)
        pltpu.sync_copy(rows_b, acc_sh.at[idx_v.at[j1]], add=True)
        return carry

    lax.fori_loop(0, _SCI // 2, body, 0)
    if _SCI % 2:
        j_last = _SCI - 1
        wait_load(j_last, rows_v, sem_a)
        pltpu.sync_copy(rows_v, acc_sh.at[idx_v.at[j_last]], add=True)
    plsc.subcore_barrier()
    pltpu.sync_copy(acc_sh.at[pl.ds(sid * _NPT, _NPT)],
                    out_hbm.at[cid, pl.ds(sid * _NPT, _NPT)])


def _mlp_body(hi_ref, hj_ref, ea_ref, w1a_ref, w1b_ref, w1c_ref, b1_ref,
              w2_ref, b2_ref, out_ref):
    acc = jnp.dot(hi_ref[...], w1a_ref[...], preferred_element_type=jnp.float32)
    acc += jnp.dot(hj_ref[...], w1b_ref[...], preferred_element_type=jnp.float32)
    acc += jnp.dot(ea_ref[...], w1c_ref[...], preferred_element_type=jnp.float32)
    acc += b1_ref[...]
    s = acc * jax.nn.sigmoid(acc)
    out = jnp.dot(s, w2_ref[...], preferred_element_type=jnp.float32)
    out_ref[...] = out + b2_ref[...]


@jax.jit
def _mlp(h_i, h_j, ea, w1, b1, w2, b2):
    w1a = w1[:NODE_DIM].astype(h_i.dtype)
    w1b = w1[NODE_DIM:2 * NODE_DIM].astype(h_j.dtype)
    w1c = w1[2 * NODE_DIM:]
    grid = (E // EBLK,)
    full = lambda shape: pl.BlockSpec(shape, lambda i: (0,) * len(shape))
    return pl.pallas_call(
        _mlp_body,
        grid=grid,
        in_specs=[
            pl.BlockSpec((EBLK, NODE_DIM), lambda i: (i, 0)),
            pl.BlockSpec((EBLK, NODE_DIM), lambda i: (i, 0)),
            pl.BlockSpec((EBLK, EDGE_DIM), lambda i: (i, 0)),
            full((NODE_DIM, HIDDEN)),
            full((NODE_DIM, HIDDEN)),
            full((EDGE_DIM, HIDDEN)),
            full((1, HIDDEN)),
            full((HIDDEN, NODE_DIM)),
            full((1, NODE_DIM)),
        ],
        out_specs=pl.BlockSpec((EBLK, NODE_DIM), lambda i: (i, 0)),
        out_shape=jax.ShapeDtypeStruct((E, NODE_DIM), jnp.float32),
    )(h_i, h_j, ea, w1a, w1b, w1c, b1.reshape(1, HIDDEN), w2,
      b2.reshape(1, NODE_DIM))


_RBF_STEP = 5.0 / 7.0
_RBF_COEFF = -0.5 / _RBF_STEP ** 2


def _ea_body(d2_ref, t_ref, st_ref, out_ref):
    dist = jnp.sqrt(d2_ref[...] + 1e-12)
    off = (jax.lax.broadcasted_iota(jnp.int32, (1, EDGE_DIM // 2), 1)
           .astype(jnp.float32) * _RBF_STEP)
    rbf = jnp.exp(_RBF_COEFF * jnp.square(dist - off))
    t = t_ref[...]
    s0 = st_ref[0:1, :]
    s1 = st_ref[1:2, :]
    kind = s0 + t * (s1 - s0)
    out_ref[...] = jnp.concatenate([rbf, kind], axis=1)


@jax.jit
def _edge_attr(d2, t, st_pad):
    return pl.pallas_call(
        _ea_body,
        grid=(E // EBLK,),
        in_specs=[
            pl.BlockSpec((EBLK, 1), lambda i: (i, 0)),
            pl.BlockSpec((EBLK, 1), lambda i: (i, 0)),
            pl.BlockSpec((8, EDGE_DIM // 2), lambda i: (0, 0)),
        ],
        out_specs=pl.BlockSpec((EBLK, EDGE_DIM), lambda i: (i, 0)),
        out_shape=jax.ShapeDtypeStruct((E, EDGE_DIM), jnp.float32),
    )(d2, t, st_pad)


_PBLK = 2000


def _pool_body(b_ref, h_ref, fcw_ref, fcb_ref, out_ref, sums, cnts):
    i = pl.program_id(0)

    @pl.when(i == 0)
    def _():
        sums[...] = jnp.zeros_like(sums)
        cnts[...] = jnp.zeros_like(cnts)

    b = b_ref[pl.ds(i * _PBLK, _PBLK), :]
    g = jax.lax.broadcasted_iota(jnp.int32, (1, N_GRAPHS), 1).astype(jnp.float32)
    s = (b == g).astype(jnp.float32)
    dims = (((0,), (0,)), ((), ()))
    sums[...] += jax.lax.dot_general(s, h_ref[...], dims,
                                     preferred_element_type=jnp.float32)
    cnts[...] += jax.lax.dot_general(
        s, jnp.ones_like(h_ref[...]), dims,
        preferred_element_type=jnp.float32)

    @pl.when(i == (N // _PBLK) - 1)
    def _():
        pooled = sums[...] / jnp.maximum(cnts[...], 1.0)
        out_ref[...] = jnp.dot(pooled, fcw_ref[...],
                               preferred_element_type=jnp.float32) + fcb_ref[...]


@jax.jit
def _pool_fc(batch_f, h, fc_w, fc_b):
    return pl.pallas_call(
        _pool_body,
        grid=(N // _PBLK,),
        in_specs=[
            pl.BlockSpec((N, 1), lambda i: (0, 0)),
            pl.BlockSpec((_PBLK, NODE_DIM), lambda i: (i, 0)),
            pl.BlockSpec((NODE_DIM, 1), lambda i: (0, 0)),
            pl.BlockSpec((1, 1), lambda i: (0, 0)),
        ],
        out_specs=pl.BlockSpec((N_GRAPHS, 1), lambda i: (0, 0)),
        out_shape=jax.ShapeDtypeStruct((N_GRAPHS, 1), jnp.float32),
        scratch_shapes=[
            pltpu.VMEM((N_GRAPHS, NODE_DIM), jnp.float32),
            pltpu.VMEM((N_GRAPHS, NODE_DIM), jnp.float32),
        ],
    )(batch_f, h, fc_w, fc_b)


_NPAD = 10240


def kernel(atoms, edge_index, coordinates, is_receptor, batch, node_table,
           subunit_table, W1s, b1s, W2s, b2s, fc_w, fc_b):
    src = edge_index[0]
    dst = edge_index[1]
    atoms_p = jnp.concatenate(
        [atoms, jnp.zeros((_NPAD - N,), atoms.dtype)]).astype(jnp.int32)
    h = _make_gather1(_NPAD, NODE_DIM, 320)(node_table, atoms_p)[:N]
    geo = jnp.concatenate(
        [coordinates.T.reshape(-1),
         is_receptor.astype(jnp.float32)])
    d2, t = _sc_edge_geo(geo, src, dst)
    st_pad = jnp.zeros((8, EDGE_DIM // 2), jnp.float32).at[:2].set(subunit_table)
    edge_attr = _edge_attr(d2[:, None], t[:, None], st_pad)
    dst3 = dst.reshape(_NW, _SCI, _SCC)
    zeros_n = jnp.zeros((_NP, NODE_DIM), jnp.float32)
    gather_h = _make_gather2(E, NODE_DIM, 200)
    for i in range(N_LAYERS):
        h_j, h_i = gather_h(h, src, dst)
        m = _mlp(h_i, h_j, edge_attr, W1s[i], b1s[i], W2s[i], b2s[i])
        acc = _sc_scatter_add(m, dst3, zeros_n)
        h = jax.nn.relu(h + acc[0, :N] + acc[1, :N])
    out = _pool_fc(batch.astype(jnp.float32)[:, None], h, fc_w,
                   fc_b.reshape(1, 1))
    return out[:, 0]


# speedup vs baseline: 5.9740x; 1.0033x over previous
import functools

import jax
import jax.numpy as jnp
from jax import lax
from jax.experimental import pallas as pl
from jax.experimental.pallas import tpu as pltpu
from jax.experimental.pallas import tpu_sc as plsc

N = 10000
E = 320000
NODE_DIM = 128
EDGE_DIM = 16
HIDDEN = 2 * NODE_DIM
N_LAYERS = 3
N_GRAPHS = 16

EBLK = 2000

_NC = 2
_NS = 16
_NW = _NC * _NS
_EPW = E // _NW
_GC = 400

_sc_mesh = lambda: plsc.VectorSubcoreMesh(core_axis_name="c",
                                          subcore_axis_name="s")


@functools.cache
def _make_gather2(n_rows, width, chunk, dtype=jnp.float32):
    per_w = n_rows // _NW
    n_it = per_w // chunk

    @functools.partial(
        pl.kernel,
        out_type=(jax.ShapeDtypeStruct((n_rows, width), dtype),
                  jax.ShapeDtypeStruct((n_rows, width), dtype)),
        mesh=_sc_mesh(),
        scratch_types=[
            pltpu.VMEM((chunk,), jnp.int32),
            pltpu.VMEM((chunk, width), dtype),
            pltpu.VMEM((chunk,), jnp.int32),
            pltpu.VMEM((chunk, width), dtype),
            pltpu.VMEM((chunk,), jnp.int32),
            pltpu.VMEM((chunk, width), dtype),
            pltpu.VMEM((chunk,), jnp.int32),
            pltpu.VMEM((chunk, width), dtype),
            pltpu.SemaphoreType.DMA,
            pltpu.SemaphoreType.DMA,
            pltpu.SemaphoreType.DMA,
            pltpu.SemaphoreType.DMA,
            pltpu.SemaphoreType.DMA,
            pltpu.SemaphoreType.DMA,
            pltpu.SemaphoreType.DMA,
            pltpu.SemaphoreType.DMA,
        ],
    )
    def gather2(tab_hbm, src_hbm, dst_hbm, oj_hbm, oi_hbm,
                sidx_a, srows_a, didx_a, drows_a,
                sidx_b, srows_b, didx_b, drows_b,
                sem_sa, sem_da, sem_sb, sem_db,
                sem_wja, sem_wia, sem_wjb, sem_wib):
        wid = lax.axis_index("s") * _NC + lax.axis_index("c")
        base0 = wid * per_w

        def wait_writes(srows, drows, sem_wj, sem_wi):
            pltpu.make_async_copy(srows, oj_hbm.at[pl.ds(base0, chunk)],
                                  sem_wj).wait()
            pltpu.make_async_copy(drows, oi_hbm.at[pl.ds(base0, chunk)],
                                  sem_wi).wait()

        def process(c, first, sidx, didx, srows, drows, sem_s, sem_d,
                    sem_wj, sem_wi):
            base = base0 + c * chunk

            @pl.when(jnp.logical_not(first))
            def _():
                wait_writes(srows, drows, sem_wj, sem_wi)

            pltpu.sync_copy(src_hbm.at[pl.ds(base, chunk)], sidx)
            pltpu.sync_copy(dst_hbm.at[pl.ds(base, chunk)], didx)
            cp_s = pltpu.async_copy(tab_hbm.at[sidx], srows, sem_s)
            cp_d = pltpu.async_copy(tab_hbm.at[didx], drows, sem_d)
            cp_s.wait()
            cp_d.wait()
            pltpu.async_copy(srows, oj_hbm.at[pl.ds(base, chunk)], sem_wj)
            pltpu.async_copy(drows, oi_hbm.at[pl.ds(base, chunk)], sem_wi)

        def body(k, carry):
            process(2 * k, k == 0, sidx_a, didx_a, srows_a, drows_a,
                    sem_sa, sem_da, sem_wja, sem_wia)
            process(2 * k + 1, k == 0, sidx_b, didx_b, srows_b, drows_b,
                    sem_sb, sem_db, sem_wjb, sem_wib)
            return carry

        lax.fori_loop(0, n_it // 2, body, 0)
        if n_it % 2:
            process(n_it - 1, n_it == 1, sidx_a, didx_a, srows_a, drows_a,
                    sem_sa, sem_da, sem_wja, sem_wia)
        wait_writes(srows_a, drows_a, sem_wja, sem_wia)
        if n_it > 1:
            wait_writes(srows_b, drows_b, sem_wjb, sem_wib)

    return gather2


_GEOC = 2000


@functools.partial(
    pl.kernel,
    out_type=(jax.ShapeDtypeStruct((E,), jnp.float32),
              jax.ShapeDtypeStruct((E,), jnp.float32)),
    mesh=_sc_mesh(),
    scratch_types=[
        pltpu.VMEM((4 * N,), jnp.float32),
        pltpu.VMEM((_GEOC,), jnp.int32),
        pltpu.VMEM((_GEOC,), jnp.int32),
        pltpu.VMEM((_GEOC,), jnp.float32),
        pltpu.VMEM((_GEOC,), jnp.float32),
    ],
    compiler_params=pltpu.CompilerParams(needs_layout_passes=False),
)
def _sc_edge_geo(geo_hbm, src_hbm, dst_hbm, d2_hbm, t_hbm,
                 geo_v, sidx, didx, d2_v, t_v):
    wid = lax.axis_index("s") * _NC + lax.axis_index("c")
    base0 = wid * _EPW
    pltpu.sync_copy(geo_hbm, geo_v)

    def chunk_body(it, carry):
        base = base0 + it * _GEOC
        pltpu.sync_copy(src_hbm.at[pl.ds(base, _GEOC)], sidx)
        pltpu.sync_copy(dst_hbm.at[pl.ds(base, _GEOC)], didx)

        def grp_body(g):
            sl = pl.ds(g * 16, 16)
            sv = sidx[sl]
            dv = didx[sl]
            dx = (plsc.load_gather(geo_v, [sv])
                  - plsc.load_gather(geo_v, [dv]))
            dy = (plsc.load_gather(geo_v, [sv + N])
                  - plsc.load_gather(geo_v, [dv + N]))
            dz = (plsc.load_gather(geo_v, [sv + 2 * N])
                  - plsc.load_gather(geo_v, [dv + 2 * N]))
            rs = plsc.load_gather(geo_v, [sv + 3 * N])
            rd = plsc.load_gather(geo_v, [dv + 3 * N])
            d2_v[sl] = dx * dx + dy * dy + dz * dz
            t_v[sl] = jnp.where(rs != rd, 1.0, 0.0).astype(jnp.float32)

        plsc.parallel_loop(0, _GEOC // 16, 1, unroll=8)(grp_body)
        pltpu.sync_copy(d2_v, d2_hbm.at[pl.ds(base, _GEOC)])
        pltpu.sync_copy(t_v, t_hbm.at[pl.ds(base, _GEOC)])
        return carry

    lax.fori_loop(0, _EPW // _GEOC, chunk_body, 0)


@functools.cache
def _make_gather1(n_rows, width, chunk):
    per_w = n_rows // _NW

    @functools.partial(
        pl.kernel,
        out_type=jax.ShapeDtypeStruct((n_rows, width), jnp.float32),
        mesh=_sc_mesh(),
        scratch_types=[
            pltpu.VMEM((chunk,), jnp.int32),
            pltpu.VMEM((chunk, width), jnp.float32),
            pltpu.SemaphoreType.DMA,
        ],
    )
    def gather1(tab_hbm, idx_hbm, out_hbm, vidx, vrows, sem):
        wid = lax.axis_index("s") * _NC + lax.axis_index("c")
        base0 = wid * per_w

        def body(it, carry):
            base = base0 + it * chunk
            pltpu.sync_copy(idx_hbm.at[pl.ds(base, chunk)], vidx)
            pltpu.async_copy(tab_hbm.at[vidx], vrows, sem).wait()
            pltpu.sync_copy(vrows, out_hbm.at[pl.ds(base, chunk)])
            return carry

        lax.fori_loop(0, per_w // chunk, body, 0)

    return gather1


_SCC = 80
_SCI = _EPW // _SCC
_NP = 10240
_NPT = _NP // _NS


@functools.partial(
    pl.kernel,
    out_type=jax.ShapeDtypeStruct((_NC, _NP, NODE_DIM), jnp.float32),
    mesh=_sc_mesh(),
    scratch_types=[
        pltpu.VMEM((_SCI, _SCC), jnp.int32),
        pltpu.VMEM((_SCC, NODE_DIM), jnp.float32),
        pltpu.VMEM((_SCC, NODE_DIM), jnp.float32),
        pltpu.VMEM_SHARED((_NP, NODE_DIM), jnp.float32),
        pltpu.SemaphoreType.DMA,
        pltpu.SemaphoreType.DMA,
    ],
)
def _sc_scatter_add(m_hbm, dst3_hbm, zeros_hbm, out_hbm, idx_v, rows_v, rows_b,
                    acc_sh, sem_a, sem_b):
    cid = lax.axis_index("c")
    sid = lax.axis_index("s")
    wid = sid * _NC + cid
    base0 = wid * _EPW
    pltpu.sync_copy(zeros_hbm.at[pl.ds(sid * _NPT, _NPT)],
                    acc_sh.at[pl.ds(sid * _NPT, _NPT)])
    pltpu.sync_copy(dst3_hbm.at[wid], idx_v)
    plsc.subcore_barrier()

    def load(j, rows, sem):
        pltpu.async_copy(m_hbm.at[pl.ds(base0 + j * _SCC, _SCC)], rows, sem)

    def wait_load(j, rows, sem):
        pltpu.make_async_copy(m_hbm.at[pl.ds(base0 + j * _SCC, _SCC)],
                              rows, sem).wait()

    load(0, rows_v, sem_a)

    def body(k, carry):
        j0 = 2 * k
        j1 = j0 + 1
        load(j1, rows_b, sem_b)
        wait_load(j0, rows_v, sem_a)
        pltpu.sync_copy(rows_v, acc_sh.at[idx_v.at[j0]], add=True)

        @pl.when(j0 + 2 < _SCI)
        def _():
            load(j0 + 2, rows_v, sem_a)

        wait_load(j1, rows_b, sem_b)
        pltpu.sync_copy(rows_b, acc_sh.at[idx_v.at[j1]], add=True)
        return carry

    lax.fori_loop(0, _SCI // 2, body, 0)
    if _SCI % 2:
        j_last = _SCI - 1
        wait_load(j_last, rows_v, sem_a)
        pltpu.sync_copy(rows_v, acc_sh.at[idx_v.at[j_last]], add=True)
    plsc.subcore_barrier()
    pltpu.sync_copy(acc_sh.at[pl.ds(sid * _NPT, _NPT)],
                    out_hbm.at[cid, pl.ds(sid * _NPT, _NPT)])


def _mlp_body(hi_ref, hj_ref, ea_ref, w1a_ref, w1b_ref, w1c_ref, b1_ref,
              w2_ref, b2_ref, out_ref):
    acc = jnp.dot(hi_ref[...], w1a_ref[...], preferred_element_type=jnp.float32)
    acc += jnp.dot(hj_ref[...], w1b_ref[...], preferred_element_type=jnp.float32)
    acc += jnp.dot(ea_ref[...], w1c_ref[...], preferred_element_type=jnp.float32)
    acc += b1_ref[...]
    s = acc * jax.nn.sigmoid(acc)
    out = jnp.dot(s, w2_ref[...], preferred_element_type=jnp.float32)
    out_ref[...] = out + b2_ref[...]


@jax.jit
def _mlp(h_i, h_j, ea, w1, b1, w2, b2):
    w1a = w1[:NODE_DIM].astype(h_i.dtype)
    w1b = w1[NODE_DIM:2 * NODE_DIM].astype(h_j.dtype)
    w1c = w1[2 * NODE_DIM:]
    grid = (E // EBLK,)
    full = lambda shape: pl.BlockSpec(shape, lambda i: (0,) * len(shape))
    return pl.pallas_call(
        _mlp_body,
        grid=grid,
        in_specs=[
            pl.BlockSpec((EBLK, NODE_DIM), lambda i: (i, 0)),
            pl.BlockSpec((EBLK, NODE_DIM), lambda i: (i, 0)),
            pl.BlockSpec((EBLK, EDGE_DIM), lambda i: (i, 0)),
            full((NODE_DIM, HIDDEN)),
            full((NODE_DIM, HIDDEN)),
            full((EDGE_DIM, HIDDEN)),
            full((1, HIDDEN)),
            full((HIDDEN, NODE_DIM)),
            full((1, NODE_DIM)),
        ],
        out_specs=pl.BlockSpec((EBLK, NODE_DIM), lambda i: (i, 0)),
        out_shape=jax.ShapeDtypeStruct((E, NODE_DIM), jnp.float32),
    )(h_i, h_j, ea, w1a, w1b, w1c, b1.reshape(1, HIDDEN), w2,
      b2.reshape(1, NODE_DIM))


_RBF_STEP = 5.0 / 7.0
_RBF_COEFF = -0.5 / _RBF_STEP ** 2


def _ea_body(d2_ref, t_ref, st_ref, out_ref):
    dist = jnp.sqrt(d2_ref[...] + 1e-12)
    off = (jax.lax.broadcasted_iota(jnp.int32, (1, EDGE_DIM // 2), 1)
           .astype(jnp.float32) * _RBF_STEP)
    rbf = jnp.exp(_RBF_COEFF * jnp.square(dist - off))
    t = t_ref[...]
    s0 = st_ref[0:1, :]
    s1 = st_ref[1:2, :]
    kind = s0 + t * (s1 - s0)
    out_ref[...] = jnp.concatenate([rbf, kind], axis=1)


@jax.jit
def _edge_attr(d2, t, st_pad):
    return pl.pallas_call(
        _ea_body,
        grid=(E // EBLK,),
        in_specs=[
            pl.BlockSpec((EBLK, 1), lambda i: (i, 0)),
            pl.BlockSpec((EBLK, 1), lambda i: (i, 0)),
            pl.BlockSpec((8, EDGE_DIM // 2), lambda i: (0, 0)),
        ],
        out_specs=pl.BlockSpec((EBLK, EDGE_DIM), lambda i: (i, 0)),
        out_shape=jax.ShapeDtypeStruct((E, EDGE_DIM), jnp.float32),
    )(d2, t, st_pad)


_PBLK = 2000


def _pool_body(b_ref, h_ref, fcw_ref, fcb_ref, out_ref, sums, cnts):
    i = pl.program_id(0)

    @pl.when(i == 0)
    def _():
        sums[...] = jnp.zeros_like(sums)
        cnts[...] = jnp.zeros_like(cnts)

    b = b_ref[pl.ds(i * _PBLK, _PBLK), :]
    g = jax.lax.broadcasted_iota(jnp.int32, (1, N_GRAPHS), 1).astype(jnp.float32)
    s = (b == g).astype(jnp.float32)
    dims = (((0,), (0,)), ((), ()))
    sums[...] += jax.lax.dot_general(s, h_ref[...], dims,
                                     preferred_element_type=jnp.float32)
    cnts[...] += jax.lax.dot_general(
        s, jnp.ones_like(h_ref[...]), dims,
        preferred_element_type=jnp.float32)

    @pl.when(i == (N // _PBLK) - 1)
    def _():
        pooled = sums[...] / jnp.maximum(cnts[...], 1.0)
        out_ref[...] = jnp.dot(pooled, fcw_ref[...],
                               preferred_element_type=jnp.float32) + fcb_ref[...]


@jax.jit
def _pool_fc(batch_f, h, fc_w, fc_b):
    return pl.pallas_call(
        _pool_body,
        grid=(N // _PBLK,),
        in_specs=[
            pl.BlockSpec((N, 1), lambda i: (0, 0)),
            pl.BlockSpec((_PBLK, NODE_DIM), lambda i: (i, 0)),
            pl.BlockSpec((NODE_DIM, 1), lambda i: (0, 0)),
            pl.BlockSpec((1, 1), lambda i: (0, 0)),
        ],
        out_specs=pl.BlockSpec((N_GRAPHS, 1), lambda i: (0, 0)),
        out_shape=jax.ShapeDtypeStruct((N_GRAPHS, 1), jnp.float32),
        scratch_shapes=[
            pltpu.VMEM((N_GRAPHS, NODE_DIM), jnp.float32),
            pltpu.VMEM((N_GRAPHS, NODE_DIM), jnp.float32),
        ],
    )(batch_f, h, fc_w, fc_b)


_NPAD = 10240


def kernel(atoms, edge_index, coordinates, is_receptor, batch, node_table,
           subunit_table, W1s, b1s, W2s, b2s, fc_w, fc_b):
    src = edge_index[0]
    dst = edge_index[1]
    atoms_p = jnp.concatenate(
        [atoms, jnp.zeros((_NPAD - N,), atoms.dtype)]).astype(jnp.int32)
    h = _make_gather1(_NPAD, NODE_DIM, 320)(node_table, atoms_p)[:N]
    geo = jnp.concatenate(
        [coordinates.T.reshape(-1),
         is_receptor.astype(jnp.float32)])
    d2, t = _sc_edge_geo(geo, src, dst)
    st_pad = jnp.zeros((8, EDGE_DIM // 2), jnp.float32).at[:2].set(subunit_table)
    edge_attr = _edge_attr(d2[:, None], t[:, None], st_pad)
    dst3 = dst.reshape(_NW, _SCI, _SCC)
    zeros_n = jnp.zeros((_NP, NODE_DIM), jnp.float32)
    gather_h = _make_gather2(E, NODE_DIM, 200)
    for i in range(N_LAYERS):
        h_j, h_i = gather_h(h, src, dst)
        m = _mlp(h_i, h_j, edge_attr, W1s[i], b1s[i], W2s[i], b2s[i])
        acc = _sc_scatter_add(m, dst3, zeros_n)
        h = jax.nn.relu(h + acc[0, :N] + acc[1, :N])
    out = _pool_fc(batch.astype(jnp.float32)[:, None], h, fc_w,
                   fc_b.reshape(1, 1))
    return out[:, 0]
